# baseline (speedup 1.0000x reference)
import functools

import jax
import jax.numpy as jnp
from jax import lax
from jax.experimental import pallas as pl
from jax.experimental.pallas import tpu as pltpu
from jax.experimental.pallas import tpu_sc as plsc

VOCAB = 1000000
D = 32
B = 16384
NC = 2
NS = 16
NW = NC * NS
BPW = B // NW
WAVE = 16
NWAVES = BPW // WAVE

_mesh = plsc.VectorSubcoreMesh(core_axis_name="c", subcore_axis_name="s")


@functools.partial(
    pl.kernel,
    mesh=_mesh,
    compiler_params=pltpu.CompilerParams(needs_layout_passes=False),
    out_type=[
        jax.ShapeDtypeStruct((D, B), jnp.float32),
        jax.ShapeDtypeStruct((16,), jnp.float32),
    ],
    scratch_types=[
        pltpu.VMEM((BPW,), jnp.int32),
        pltpu.VMEM((WAVE * D, 128), jnp.float32),
        pltpu.VMEM((D, BPW), jnp.float32),
        pltpu.VMEM((16,), jnp.float32),
        pltpu.SemaphoreType.DMA,
    ],
)
def _policy_gather(idx_hbm, xt_hbm, sig_hbm, out_hbm, sig_out_hbm,
                   idx_v, ring_v, out_v, sig_v, sem):
    wid = lax.axis_index("s") * NC + lax.axis_index("c")

    pltpu.sync_copy(idx_hbm.at[pl.ds(BPW * wid, BPW)], idx_v)

    jlane = lax.broadcasted_iota(jnp.int32, (16,), 0)

    @pl.loop(0, NWAVES)
    def _wave(g):
        idx16 = idx_v[pl.ds(g * WAVE, WAVE)]
        pagev = (idx16 >> 7) << 7
        lanev = idx16 & 127

        copies = []
        for l in range(WAVE):
            off = pl.multiple_of(pagev[l], 128)
            copies.append(pltpu.async_copy(
                xt_hbm.at[:, pl.ds(off, 128)],
                ring_v.at[pl.ds(l * D, D), :],
                sem))
        for c in copies:
            c.wait()

        for l in range(WAVE):
            lane = jnp.full((16,), 0, jnp.int32) + lanev[l]
            col = jnp.full((16,), 0, jnp.int32) + (g * WAVE + l)
            r0 = jlane + l * D
            v0 = plsc.load_gather(ring_v, [r0, lane])
            plsc.store_scatter(out_v, [jlane, col], v0)
            v1 = plsc.load_gather(ring_v, [r0 + 16, lane])
            plsc.store_scatter(out_v, [jlane + 16, col], v1)

    pltpu.sync_copy(out_v, out_hbm.at[:, pl.ds(BPW * wid, BPW)])

    @pl.when(wid == 0)
    def _clip_sigma():
        pltpu.sync_copy(sig_hbm, sig_v)
        v = sig_v[...]
        sig_v[...] = jnp.minimum(jnp.maximum(v, jnp.float32(-2.5)),
                                 jnp.float32(0.0))
        pltpu.sync_copy(sig_v, sig_out_hbm)


def kernel(indices, X, log_sigma):
    xt = X.T
    sig16 = jnp.broadcast_to(log_sigma, (16,))
    outt, sig = _policy_gather(indices, xt, sig16)
    return outt.T, sig[:1]


# speedup vs baseline: 1.3239x; 1.3239x over previous
import functools

import jax
import jax.numpy as jnp
from jax import lax
from jax.experimental import pallas as pl
from jax.experimental.pallas import tpu as pltpu
from jax.experimental.pallas import tpu_sc as plsc

VOCAB = 1000000
D = 32
B = 16384
NC = 2
NS = 16
NW = NC * NS
BPW = B // NW
WAVE = 8
NWAVES = BPW // WAVE
BW = 128

_mesh = plsc.VectorSubcoreMesh(core_axis_name="c", subcore_axis_name="s")


@functools.partial(
    pl.kernel,
    mesh=_mesh,
    compiler_params=pltpu.CompilerParams(needs_layout_passes=False),
    out_type=[
        jax.ShapeDtypeStruct((D, B), jnp.float32),
        jax.ShapeDtypeStruct((16,), jnp.float32),
    ],
    scratch_types=[
        pltpu.VMEM((BPW + 16,), jnp.int32),
        pltpu.VMEM((2 * WAVE * D, BW), jnp.float32),
        pltpu.VMEM((D, BPW), jnp.float32),
        pltpu.VMEM((16,), jnp.float32),
        pltpu.SemaphoreType.DMA,
    ],
)
def _policy_gather(idx_hbm, xt_hbm, sig_hbm, out_hbm, sig_out_hbm,
                   idx_v, ring_v, out_v, sig_v, sem):
    wid = lax.axis_index("s") * NC + lax.axis_index("c")

    pltpu.sync_copy(idx_hbm.at[pl.ds(BPW * wid, BPW)], idx_v.at[pl.ds(0, BPW)])

    jlane = lax.broadcasted_iota(jnp.int32, (16,), 0)

    def fire(g, buf):
        idx16 = idx_v[pl.ds(g * WAVE, 16)]
        pagev = (idx16 >> 7) << 7
        for l in range(WAVE):
            off = pl.multiple_of(pagev[l], BW)
            pltpu.async_copy(
                xt_hbm.at[:, pl.ds(off, BW)],
                ring_v.at[pl.ds((buf * WAVE + l) * D, D), :],
                sem)

    def drain_and_extract(g, buf):
        for l in range(WAVE):
            pltpu.make_async_copy(
                xt_hbm.at[:, pl.ds(0, BW)],
                ring_v.at[pl.ds((buf * WAVE + l) * D, D), :],
                sem).wait()
        idx16 = idx_v[pl.ds(g * WAVE, 16)]
        lanev = idx16 & (BW - 1)
        base = buf * WAVE * D
        for l in range(WAVE):
            lane = jnp.full((16,), 0, jnp.int32) + lanev[l]
            col = jnp.full((16,), 0, jnp.int32) + (g * WAVE + l)
            r0 = jlane + (base + l * D)
            v0 = plsc.load_gather(ring_v, [r0, lane])
            plsc.store_scatter(out_v, [jlane, col], v0)
            v1 = plsc.load_gather(ring_v, [r0 + 16, lane])
            plsc.store_scatter(out_v, [jlane + 16, col], v1)

    fire(0, 0)

    @pl.loop(0, NWAVES // 2)
    def _pair(h):
        g0 = h * 2
        pl.when(g0 + 1 < NWAVES)(lambda: fire(g0 + 1, 1))
        drain_and_extract(g0, 0)
        pl.when(g0 + 2 < NWAVES)(lambda: fire(g0 + 2, 0))
        drain_and_extract(g0 + 1, 1)

    pltpu.sync_copy(out_v, out_hbm.at[:, pl.ds(BPW * wid, BPW)])

    @pl.when(wid == 0)
    def _clip_sigma():
        pltpu.sync_copy(sig_hbm, sig_v)
        v = sig_v[...]
        sig_v[...] = jnp.minimum(jnp.maximum(v, jnp.float32(-2.5)),
                                 jnp.float32(0.0))
        pltpu.sync_copy(sig_v, sig_out_hbm)


def kernel(indices, X, log_sigma):
    xt = X.T
    sig16 = jnp.broadcast_to(log_sigma, (16,))
    outt, sig = _policy_gather(indices, xt, sig16)
    return outt.T, sig[:1]
